# Initial kernel scaffold; baseline (speedup 1.0000x reference)
import jax
import jax.numpy as jnp
from jax.experimental import pallas as pl


def kernel(ufeats, mfeats, edges_src, edges_dst, W_self_movie, W_neigh_user, W_self_user, W_neigh_movie, b_u2m, b_m2u, decoders):
    raise NotImplementedError("write your pallas kernel here")


# speedup vs baseline: 2.4871x; 2.4871x over previous
import functools

import jax
import jax.numpy as jnp
from jax import lax
from jax.experimental import pallas as pl
from jax.experimental.pallas import tpu as pltpu
from jax.experimental.pallas import tpu_sc as plsc

_R = 5
_N = 5000
_E = 64000
_DF = 128
_H = 8
_RH = _R * _H
_PW = 16
_RW = 48
_CH = 80
_ROWS = _R * _E // _CH
_NSC = 16
_ACC = _R * _N
_ACCP = 25600
_NB = 1000


def _project_body(uf_ref, mf_ref, wnu_ref, wnm_ref, wsm_ref, wsu_ref,
                  bm_ref, bu_ref, pu_ref, pm_ref, sm_ref, su_ref):
    u = uf_ref[...]
    m = mf_ref[...]
    pu = jnp.dot(u, wnu_ref[...], preferred_element_type=jnp.float32)
    pm = jnp.dot(m, wnm_ref[...], preferred_element_type=jnp.float32)
    sm = jnp.dot(m, wsm_ref[...], preferred_element_type=jnp.float32) + bm_ref[...]
    su = jnp.dot(u, wsu_ref[...], preferred_element_type=jnp.float32) + bu_ref[...]
    ones = jnp.ones((_NB, 1), jnp.float32)
    z7 = jnp.zeros((_NB, _PW - _H - 1), jnp.float32)
    for r in range(_R):
        pu_ref[r] = jnp.concatenate([pu[:, _H * r:_H * r + _H], ones, z7], axis=1)
        pm_ref[r] = jnp.concatenate([pm[:, _H * r:_H * r + _H], ones, z7], axis=1)
        sm_ref[r] = sm[:, _H * r:_H * r + _H]
        su_ref[r] = su[:, _H * r:_H * r + _H]


def _tc_project(uf, mf, wnu, wnm, wsm, wsu, bm, bu):
    nblk = _N // _NB
    w_spec = pl.BlockSpec((_DF, _RH), lambda i: (0, 0))
    b_spec = pl.BlockSpec((1, _RH), lambda i: (0, 0))
    f_spec = pl.BlockSpec((_NB, _DF), lambda i: (i, 0))
    p_spec = pl.BlockSpec((_R, _NB, _PW), lambda i: (0, i, 0))
    s_spec = pl.BlockSpec((_R, _NB, _H), lambda i: (0, i, 0))
    return pl.pallas_call(
        _project_body,
        grid=(nblk,),
        in_specs=[f_spec, f_spec, w_spec, w_spec, w_spec, w_spec, b_spec, b_spec],
        out_specs=[p_spec, p_spec, s_spec, s_spec],
        out_shape=[
            jax.ShapeDtypeStruct((_R, _N, _PW), jnp.float32),
            jax.ShapeDtypeStruct((_R, _N, _PW), jnp.float32),
            jax.ShapeDtypeStruct((_R, _N, _H), jnp.float32),
            jax.ShapeDtypeStruct((_R, _N, _H), jnp.float32),
        ],
    )(uf, mf, wnu, wnm, wsm, wsu, bm, bu)


def _segsum_body(pu_h, pm_h, iu_h, im_h, out_h,
                 gi, si, rows, zbuf, acc_sh, sem):
    c = lax.axis_index("c")
    s = lax.axis_index("s")
    sbase = s * (_ACCP // _NSC)

    def zrow(i, _):
        zbuf[i] = jnp.zeros((16,), jnp.float32)
        return 0
    lax.fori_loop(0, 160, zrow, 0)

    def zchunk(k, _):
        pltpu.sync_copy(zbuf, acc_sh.at[pl.ds(sbase + k * 160, 160)])
        return 0
    lax.fori_loop(0, _ACCP // _NSC // 160, zchunk, 0)
    plsc.subcore_barrier()

    rows_per_sub = _E // _CH // _NSC

    def pipe(tbl_h, gidx_h, sidx_h):
        for r in range(_R):
            pltpu.sync_copy(gidx_h.at[r, s], gi)
            pltpu.sync_copy(sidx_h.at[r, s], si)

            def body(j, _):
                pltpu.async_copy(tbl_h.at[gi.at[j]], rows, sem).wait()
                pltpu.sync_copy(rows, acc_sh.at[si.at[j]], add=True)
                return 0
            lax.fori_loop(0, rows_per_sub, body, 0)

    @pl.when(c == 0)
    def _():
        pipe(pu_h, iu_h, im_h)

    @pl.when(c == 1)
    def _():
        pipe(pm_h, im_h, iu_h)

    plsc.subcore_barrier()
    nwb = _ACCP // _NSC
    pltpu.sync_copy(acc_sh.at[pl.ds(sbase, nwb)],
                    out_h.at[c].at[pl.ds(sbase, nwb)])


def _sc_segsum(pu2, pm2, iu, im):
    mesh = plsc.VectorSubcoreMesh(core_axis_name="c", subcore_axis_name="s")
    f = functools.partial(
        pl.kernel,
        out_type=jax.ShapeDtypeStruct((2, _ACCP, _PW), jnp.float32),
        mesh=mesh,
        scratch_types=[
            pltpu.VMEM((_E // _CH // _NSC, _CH), jnp.int32),
            pltpu.VMEM((_E // _CH // _NSC, _CH), jnp.int32),
            pltpu.VMEM((_CH, _PW), jnp.float32),
            pltpu.VMEM((160, _PW), jnp.float32),
            pltpu.VMEM_SHARED((_ACCP, _PW), jnp.float32),
            pltpu.SemaphoreType.DMA,
        ],
        compiler_params=pltpu.CompilerParams(use_tc_tiling_on_sc=False),
    )(_segsum_body)
    return f(pu2, pm2, iu, im)


def _normalize_body(acc_ref, sm_ref, su_ref, rm_ref, ru_ref):
    zpad = jnp.zeros((_NB, _RW - _RH), jnp.float32)
    for side, (s_ref, o_ref) in enumerate(((sm_ref, rm_ref), (su_ref, ru_ref))):
        cols = []
        for r in range(_R):
            a = acc_ref[side, r]
            h = a[:, 0:_H] / jnp.maximum(a[:, _H:_H + 1], 1.0)
            rst = jnp.maximum(s_ref[r] + h, 0.0)
            nrm = jnp.sqrt(jnp.sum(rst * rst, axis=1, keepdims=True))
            cols.append(rst / jnp.maximum(nrm, 1e-12))
        o_ref[...] = jnp.concatenate(cols + [zpad], axis=1)


def _tc_normalize(acc, sm, su):
    nblk = _N // _NB
    a_spec = pl.BlockSpec((2, _R, _NB, _PW), lambda i: (0, 0, i, 0))
    s_spec = pl.BlockSpec((_R, _NB, _H), lambda i: (0, i, 0))
    o_spec = pl.BlockSpec((_NB, _RW), lambda i: (i, 0))
    return pl.pallas_call(
        _normalize_body,
        grid=(nblk,),
        in_specs=[a_spec, s_spec, s_spec],
        out_specs=[o_spec, o_spec],
        out_shape=[jax.ShapeDtypeStruct((_N, _RW), jnp.float32),
                   jax.ShapeDtypeStruct((_N, _RW), jnp.float32)],
    )(acc, sm, su)


def _egather_body(ru_h, rm_h, iu_h, im_h, ue_h, me_h, gidx, rows, sem):
    wid = lax.axis_index("s") * 2 + lax.axis_index("c")
    rows_per_w = _ROWS // (2 * _NSC)
    rb = wid * rows_per_w
    for tbl_h, idx_h, out_h in ((ru_h, iu_h, ue_h), (rm_h, im_h, me_h)):
        pltpu.sync_copy(idx_h.at[wid], gidx)

        def body(j, _):
            pltpu.async_copy(tbl_h.at[gidx.at[j]], rows, sem).wait()
            pltpu.sync_copy(rows, out_h.at[pl.ds((rb + j) * _CH, _CH)])
            return 0
        lax.fori_loop(0, rows_per_w, body, 0)


def _sc_edge_gather(ru, rm, srcp, dstp):
    mesh = plsc.VectorSubcoreMesh(core_axis_name="c", subcore_axis_name="s")
    f = functools.partial(
        pl.kernel,
        out_type=[jax.ShapeDtypeStruct((_R * _E, _RW), jnp.float32),
                  jax.ShapeDtypeStruct((_R * _E, _RW), jnp.float32)],
        mesh=mesh,
        scratch_types=[
            pltpu.VMEM((_ROWS // (2 * _NSC), _CH), jnp.int32),
            pltpu.VMEM((_CH, _RW), jnp.float32),
            pltpu.SemaphoreType.DMA,
        ],
        compiler_params=pltpu.CompilerParams(use_tc_tiling_on_sc=False),
    )(_egather_body)
    return f(ru, rm, srcp, dstp)


def _decoder_body(ue_ref, me_ref, d_ref, o_ref):
    ue = ue_ref[:, 0:_RH]
    me = me_ref[:, 0:_RH]
    p = jnp.dot(ue, d_ref[...], preferred_element_type=jnp.float32)
    dots = jnp.concatenate(
        [jnp.sum(p[:, _RH * k:_RH * k + _RH] * me, axis=1, keepdims=True)
         for k in range(_R)], axis=1)
    mx = jnp.max(dots, axis=1, keepdims=True)
    ex = jnp.exp(dots - mx)
    lse = jnp.log(jnp.sum(ex, axis=1, keepdims=True))
    out = dots - mx - lse
    o_ref[...] = jnp.concatenate(
        [out, jnp.zeros((out.shape[0], 3), jnp.float32)], axis=1)


def _tc_decoder(ue, me, dcat):
    blk = 1000
    nblk = _R * _E // blk
    e_spec = pl.BlockSpec((blk, _RW), lambda i: (i, 0))
    return pl.pallas_call(
        _decoder_body,
        grid=(nblk,),
        in_specs=[e_spec, e_spec, pl.BlockSpec((_RH, _R * _RH), lambda i: (0, 0))],
        out_specs=pl.BlockSpec((blk, 8), lambda i: (i, 0)),
        out_shape=jax.ShapeDtypeStruct((_R * _E, 8), jnp.float32),
    )(ue, me, dcat)


def kernel(ufeats, mfeats, edges_src, edges_dst, W_self_movie, W_neigh_user,
           W_self_user, W_neigh_movie, b_u2m, b_m2u, decoders):
    wnu = W_neigh_user.transpose(1, 0, 2).reshape(_DF, _RH)
    wnm = W_neigh_movie.transpose(1, 0, 2).reshape(_DF, _RH)
    wsm = W_self_movie.transpose(1, 0, 2).reshape(_DF, _RH)
    wsu = W_self_user.transpose(1, 0, 2).reshape(_DF, _RH)
    bm = b_u2m.reshape(1, _RH)
    bu = b_m2u.reshape(1, _RH)
    dcat = decoders.transpose(1, 0, 2).reshape(_RH, _R * _RH)

    offs = (jnp.arange(_R, dtype=jnp.int32) * _N)[:, None]
    nps = _E // _CH // _NSC
    npw = _ROWS // (2 * _NSC)
    iu = (edges_src + offs).reshape(_R, _NSC, nps, _CH)
    im = (edges_dst + offs).reshape(_R, _NSC, nps, _CH)
    srcp = edges_src.reshape(2 * _NSC, npw, _CH)
    dstp = edges_dst.reshape(2 * _NSC, npw, _CH)

    p_u, p_m, s_m, s_u = _tc_project(ufeats, mfeats, wnu, wnm, wsm, wsu, bm, bu)
    acc = _sc_segsum(p_u.reshape(_R * _N, _PW), p_m.reshape(_R * _N, _PW),
                     iu, im)
    acc_live = acc[:, :_ACC].reshape(2, _R, _N, _PW)
    res_m, res_u = _tc_normalize(acc_live, s_m, s_u)
    ue, me = _sc_edge_gather(res_u, res_m, srcp, dstp)
    outp = _tc_decoder(ue, me, dcat)
    o = outp[:, :_R].reshape(_R, _E, _R)
    return tuple(o[r] for r in range(_R))
